# Initial kernel scaffold; baseline (speedup 1.0000x reference)
import jax
import jax.numpy as jnp
from jax.experimental import pallas as pl


def kernel(specific_features, shared_features, W, att_src, att_dst, bias):
    raise NotImplementedError("write your pallas kernel here")


# speedup vs baseline: 81.4525x; 81.4525x over previous
import jax
import jax.numpy as jnp
from jax.experimental import pallas as pl

_S = 3
_SH = 3
_NODES = _S + _SH
_NBRS = (
    (1, 2, 3, 0),
    (0, 2, 4, 1),
    (1, 0, 5, 2),
    (0, 3),
    (1, 4),
    (2, 5),
)


def _leaky_relu(v):
    return jnp.where(v >= 0, v, 0.2 * v)


def _gat_fused_kernel(spec_ref, shr_ref, w_ref, asrc_ref, adst_ref,
                      bias_ref, xout_ref, fused_ref):
    bB = spec_ref.shape[1]
    D = spec_ref.shape[2]
    H = asrc_ref.shape[0]

    x_all = jnp.concatenate(
        [spec_ref[...].reshape(_S * bB, D),
         shr_ref[...].reshape(_SH * bB, D)], axis=0)
    h = jnp.dot(x_all, w_ref[...], preferred_element_type=jnp.float32)

    a_src = []
    a_dst = []
    for hh in range(H):
        hs = h[:, hh * D:(hh + 1) * D]
        a_src.append(jnp.sum(hs * asrc_ref[hh:hh + 1, :], axis=1, keepdims=True))
        a_dst.append(jnp.sum(hs * adst_ref[hh:hh + 1, :], axis=1, keepdims=True))

    inv_h = 1.0 / H
    acc_mean = None
    for i in range(_NODES):
        srcs = _NBRS[i]
        term = None
        for hh in range(H):
            ad = a_dst[hh][i * bB:(i + 1) * bB]
            al = [_leaky_relu(a_src[hh][j * bB:(j + 1) * bB] + ad) for j in srcs]
            m = al[0]
            for a in al[1:]:
                m = jnp.maximum(m, a)
            exs = [jnp.exp(a - m) for a in al]
            den = exs[0]
            for e in exs[1:]:
                den = den + e
            inv = 1.0 / (den + 1e-16)
            for j, ex in zip(srcs, exs):
                c = (ex * inv) * h[j * bB:(j + 1) * bB, hh * D:(hh + 1) * D]
                term = c if term is None else term + c
        out_i = term * inv_h + bias_ref[...]
        out_i = jnp.where(out_i > 0, out_i, jnp.exp(jnp.minimum(out_i, 0.0)) - 1.0)
        xout_ref[:, i, :] = out_i
        acc_mean = out_i if acc_mean is None else acc_mean + out_i
    fused_ref[...] = acc_mean * (1.0 / _NODES)


def kernel(specific_features, shared_features, W, att_src, att_dst, bias):
    S, B, D = specific_features.shape
    H = att_src.shape[0]
    bB = 512
    grid = (B // bB,)
    bias2 = bias.reshape(1, D)

    x_out, fused = pl.pallas_call(
        _gat_fused_kernel,
        grid=grid,
        in_specs=[
            pl.BlockSpec((S, bB, D), lambda i: (0, i, 0)),
            pl.BlockSpec((_SH, bB, D), lambda i: (0, i, 0)),
            pl.BlockSpec((D, H * D), lambda i: (0, 0)),
            pl.BlockSpec((H, D), lambda i: (0, 0)),
            pl.BlockSpec((H, D), lambda i: (0, 0)),
            pl.BlockSpec((1, D), lambda i: (0, 0)),
        ],
        out_specs=[
            pl.BlockSpec((bB, _NODES, D), lambda i: (i, 0, 0)),
            pl.BlockSpec((bB, D), lambda i: (i, 0)),
        ],
        out_shape=[
            jax.ShapeDtypeStruct((B, _NODES, D), jnp.float32),
            jax.ShapeDtypeStruct((B, D), jnp.float32),
        ],
    )(specific_features, shared_features, W, att_src, att_dst, bias2)
    return fused, x_out
